# baseline (speedup 1.0000x reference)
import functools

import jax
import jax.numpy as jnp
from jax import lax
from jax.experimental import pallas as pl
from jax.experimental.pallas import tpu as pltpu
from jax.experimental.pallas import tpu_sc as plsc


def _build_tc_add(B, S, D, M, SBLK=128):
    grid = S // SBLK
    assert S % SBLK == 0

    def body(off_ref, pos_hbm, x_ref, o_ref, pos_buf, sem):
        j = pl.program_id(0)
        nj = pl.num_programs(0)
        off = off_ref[0]

        def start(jj, slot):
            s0 = pl.multiple_of(jnp.clip(off + jj * SBLK, 0, M - SBLK), 8)
            pltpu.make_async_copy(
                pos_hbm.at[pl.ds(s0, SBLK), :], pos_buf.at[slot], sem.at[slot]
            ).start()

        @pl.when(j == 0)
        def _():
            start(0, 0)

        @pl.when(j + 1 < nj)
        def _():
            start(j + 1, lax.rem(j + 1, 2))

        p = lax.rem(j, 2)
        pltpu.make_async_copy(
            pos_hbm.at[pl.ds(0, SBLK), :], pos_buf.at[p], sem.at[p]
        ).wait()
        o_ref[...] = x_ref[...] + pos_buf[p][None, :, :]

    return pl.pallas_call(
        body,
        grid=(grid,),
        in_specs=[
            pl.BlockSpec(memory_space=pltpu.SMEM),
            pl.BlockSpec(memory_space=pl.ANY),
            pl.BlockSpec((B, SBLK, D), lambda j: (0, j, 0)),
        ],
        out_specs=pl.BlockSpec((B, SBLK, D), lambda j: (0, j, 0)),
        out_shape=jax.ShapeDtypeStruct((B, S, D), jnp.float32),
        scratch_shapes=[
            pltpu.VMEM((2, SBLK, D), jnp.float32),
            pltpu.SemaphoreType.DMA((2,)),
        ],
    )


@jax.jit
def kernel(x, pos_table, offset):
    B, S, D = x.shape
    M = pos_table.shape[0]
    off = jnp.asarray(offset, jnp.int32).reshape(1)
    return _build_tc_add(B, S, D, M)(off, pos_table, x)


# speedup vs baseline: 1.0657x; 1.0657x over previous
import jax
import jax.numpy as jnp
from jax import lax
from jax.experimental import pallas as pl
from jax.experimental.pallas import tpu as pltpu


def _build_add(B, S, D, M, SBLK=512):
    assert S % SBLK == 0

    def body(off_ref, pos_hbm, x_ref, o_ref, pos_buf, sem):
        j = pl.program_id(0)
        nj = pl.num_programs(0)
        off = off_ref[0]

        def start(jj, slot):
            s0 = pl.multiple_of(jnp.clip(off + jj * SBLK, 0, M - SBLK), 8)
            pltpu.make_async_copy(
                pos_hbm.at[pl.ds(s0, SBLK), :], pos_buf.at[slot], sem.at[slot]
            ).start()

        @pl.when(j == 0)
        def _():
            start(0, 0)

        @pl.when(j + 1 < nj)
        def _():
            start(j + 1, lax.rem(j + 1, 2))

        p = lax.rem(j, 2)
        pltpu.make_async_copy(
            pos_hbm.at[pl.ds(0, SBLK), :], pos_buf.at[p], sem.at[p]
        ).wait()
        o_ref[...] = x_ref[...] + pos_buf[p][None, :, :]

    return pl.pallas_call(
        body,
        grid=(S // SBLK,),
        in_specs=[
            pl.BlockSpec(memory_space=pltpu.SMEM),
            pl.BlockSpec(memory_space=pl.ANY),
            pl.BlockSpec((B, SBLK, D), lambda j: (0, j, 0)),
        ],
        out_specs=pl.BlockSpec((B, SBLK, D), lambda j: (0, j, 0)),
        out_shape=jax.ShapeDtypeStruct((B, S, D), jnp.float32),
        scratch_shapes=[
            pltpu.VMEM((2, SBLK, D), jnp.float32),
            pltpu.SemaphoreType.DMA((2,)),
        ],
    )


@jax.jit
def kernel(x, pos_table, offset):
    B, S, D = x.shape
    M = pos_table.shape[0]
    off = jnp.asarray(offset, jnp.int32).reshape(1)
    return _build_add(B, S, D, M)(off, pos_table, x)


# speedup vs baseline: 1.0771x; 1.0107x over previous
import jax
import jax.numpy as jnp
from jax import lax
from jax.experimental import pallas as pl
from jax.experimental.pallas import tpu as pltpu


def _build_add(B, S, D, M, SBLK=1024, BBLK=2):
    assert S % SBLK == 0 and B % BBLK == 0
    nj, nb = S // SBLK, B // BBLK

    def body(off_ref, pos_hbm, x_ref, o_ref, pos_buf, sem):
        j = pl.program_id(0)
        bi = pl.program_id(1)
        off = off_ref[0]

        def start(jj, slot):
            s0 = pl.multiple_of(jnp.clip(off + jj * SBLK, 0, M - SBLK), 8)
            pltpu.make_async_copy(
                pos_hbm.at[pl.ds(s0, SBLK), :], pos_buf.at[slot], sem.at[slot]
            ).start()

        @pl.when(jnp.logical_and(j == 0, bi == 0))
        def _():
            start(0, 0)

        @pl.when(jnp.logical_and(bi == 0, j + 1 < nj))
        def _():
            start(j + 1, lax.rem(j + 1, 2))

        p = lax.rem(j, 2)

        @pl.when(bi == 0)
        def _():
            pltpu.make_async_copy(
                pos_hbm.at[pl.ds(0, SBLK), :], pos_buf.at[p], sem.at[p]
            ).wait()

        o_ref[...] = x_ref[...] + pos_buf[p][None, :, :]

    return pl.pallas_call(
        body,
        grid=(nj, nb),
        in_specs=[
            pl.BlockSpec(memory_space=pltpu.SMEM),
            pl.BlockSpec(memory_space=pl.ANY),
            pl.BlockSpec((BBLK, SBLK, D), lambda j, bi: (bi, j, 0)),
        ],
        out_specs=pl.BlockSpec((BBLK, SBLK, D), lambda j, bi: (bi, j, 0)),
        out_shape=jax.ShapeDtypeStruct((B, S, D), jnp.float32),
        scratch_shapes=[
            pltpu.VMEM((2, SBLK, D), jnp.float32),
            pltpu.SemaphoreType.DMA((2,)),
        ],
    )


@jax.jit
def kernel(x, pos_table, offset):
    B, S, D = x.shape
    M = pos_table.shape[0]
    off = jnp.asarray(offset, jnp.int32).reshape(1)
    return _build_add(B, S, D, M)(off, pos_table, x)


# speedup vs baseline: 1.0904x; 1.0124x over previous
import jax
import jax.numpy as jnp
from jax import lax
from jax.experimental import pallas as pl
from jax.experimental.pallas import tpu as pltpu


def _build_add(B, S, D, M, SBLK=2048, BBLK=1):
    assert S % SBLK == 0 and B % BBLK == 0
    nj, nb = S // SBLK, B // BBLK

    def body(off_ref, pos_hbm, x_ref, o_ref, pos_buf, sem):
        j = pl.program_id(0)
        bi = pl.program_id(1)
        off = off_ref[0]

        def start(jj, slot):
            s0 = pl.multiple_of(jnp.clip(off + jj * SBLK, 0, M - SBLK), 8)
            pltpu.make_async_copy(
                pos_hbm.at[pl.ds(s0, SBLK), :], pos_buf.at[slot], sem.at[slot]
            ).start()

        @pl.when(jnp.logical_and(j == 0, bi == 0))
        def _():
            start(0, 0)

        @pl.when(jnp.logical_and(bi == 0, j + 1 < nj))
        def _():
            start(j + 1, lax.rem(j + 1, 2))

        p = lax.rem(j, 2)

        @pl.when(bi == 0)
        def _():
            pltpu.make_async_copy(
                pos_hbm.at[pl.ds(0, SBLK), :], pos_buf.at[p], sem.at[p]
            ).wait()

        o_ref[...] = x_ref[...] + pos_buf[p][None, :, :]

    return pl.pallas_call(
        body,
        grid=(nj, nb),
        in_specs=[
            pl.BlockSpec(memory_space=pltpu.SMEM),
            pl.BlockSpec(memory_space=pl.ANY),
            pl.BlockSpec((BBLK, SBLK, D), lambda j, bi: (bi, j, 0)),
        ],
        out_specs=pl.BlockSpec((BBLK, SBLK, D), lambda j, bi: (bi, j, 0)),
        out_shape=jax.ShapeDtypeStruct((B, S, D), jnp.float32),
        scratch_shapes=[
            pltpu.VMEM((2, SBLK, D), jnp.float32),
            pltpu.SemaphoreType.DMA((2,)),
        ],
    )


@jax.jit
def kernel(x, pos_table, offset):
    B, S, D = x.shape
    M = pos_table.shape[0]
    off = jnp.asarray(offset, jnp.int32).reshape(1)
    return _build_add(B, S, D, M)(off, pos_table, x)
